# baseline (speedup 1.0000x reference)
import functools

import jax
import jax.numpy as jnp
from jax import lax
from jax.experimental import pallas as pl
from jax.experimental.pallas import tpu as pltpu
from jax.experimental.pallas import tpu_sc as plsc

B, N, D, H1, H2, R = 1024, 8, 512, 512, 256, 8
T = B * N
BLK = 128
NBLK = T // BLK
MT = 1024
NT = T // MT + R
P = NT * MT

_NC, _NS = 2, 16
_NW = _NC * _NS
_PW = T // _NW
_SCH = 64
_SNC = _PW // _SCH
_GCH = 128


def _leaky(x):
    return jnp.where(x >= 0, x, 0.01 * x)


def _route_body(s_ref, dest_ref, texp_ref):
    sv = s_ref[...]
    jj = lax.broadcasted_iota(jnp.int32, (BLK, BLK), 0)
    ii = lax.broadcasted_iota(jnp.int32, (BLK, BLK), 1)
    tri = (jj < ii).astype(jnp.float32)
    tj = lax.broadcasted_iota(jnp.int32, (NBLK, NBLK), 1)
    ti = lax.broadcasted_iota(jnp.int32, (NBLK, NBLK), 0)
    tri_blk = (tj < ti).astype(jnp.float32)
    lane = lax.broadcasted_iota(jnp.int32, (1, 128), 1)

    dest_acc = jnp.zeros((NBLK, BLK), jnp.float32)
    texp_acc = jnp.zeros((1, 128), jnp.int32)
    ptile = jnp.float32(0.0)
    for r in range(R):
        oh = (sv == r).astype(jnp.float32)
        excl = jnp.dot(oh, tri, preferred_element_type=jnp.float32)
        counts = jnp.sum(oh, axis=1, keepdims=True)
        base = jnp.dot(tri_blk, counts, preferred_element_type=jnp.float32)
        if r > 0:
            texp_acc = texp_acc + (lane.astype(jnp.float32) >= ptile).astype(jnp.int32)
        dest_acc = dest_acc + oh * (ptile * MT + base + excl)
        total = jnp.sum(counts)
        ptile = ptile + jnp.floor((total + (MT - 1)) * (1.0 / MT))
    dest_ref[...] = dest_acc.astype(jnp.int32)
    texp_ref[...] = texp_acc


def _group_body(texp_ref, xa_ref, xo_ref, w1a_ref, w1o_ref, b1_ref, w2_ref,
                b2_ref, out_ref):
    bf = jnp.bfloat16
    r = texp_ref[pl.program_id(0)]
    h = jnp.dot(xa_ref[...].astype(bf), w1a_ref[r],
                preferred_element_type=jnp.float32)
    h = h + jnp.dot(xo_ref[...].astype(bf), w1o_ref[r],
                    preferred_element_type=jnp.float32)
    h = _leaky(h + b1_ref[r])
    out_ref[...] = _leaky(jnp.dot(h.astype(bf), w2_ref[r],
                                  preferred_element_type=jnp.float32)
                          + b2_ref[r])


def _uidot_body(u_ref, i_ref, w1u_ref, w1i_ref, b1_ref, w2_ref, b2_ref,
                aos_ref, out_ref):
    bf = jnp.bfloat16
    h = jnp.dot(u_ref[...].astype(bf), w1u_ref[...],
                preferred_element_type=jnp.float32)
    h = h + jnp.dot(i_ref[...].astype(bf), w1i_ref[...],
                    preferred_element_type=jnp.float32)
    h = _leaky(h + b1_ref[...])
    ui = _leaky(jnp.dot(h.astype(bf), w2_ref[...],
                        preferred_element_type=jnp.float32) + b2_ref[...])
    aos = aos_ref[...].reshape(B, N, H2)
    out_ref[...] = jnp.sum(aos * ui[:, None, :], axis=-1)


def _sc_scatter_body(a_hbm, o_hbm, dest_hbm, xa_hbm, xo_hbm, idx_v, rows_a,
                     rows_o, sem_a, sem_o):
    wid = lax.axis_index("s") * _NC + lax.axis_index("c")
    base = wid * _PW
    pend_a = pend_o = None
    for c in range(_SNC):
        cb = base + c * _SCH
        pltpu.sync_copy(dest_hbm.at[pl.ds(cb, _SCH)], idx_v.at[c])
        if pend_a is not None:
            pend_a.wait()
        pltpu.sync_copy(a_hbm.at[pl.ds(cb, _SCH)], rows_a)
        pend_a = pltpu.async_copy(rows_a, xa_hbm.at[idx_v.at[c]], sem_a)
        if pend_o is not None:
            pend_o.wait()
        pltpu.sync_copy(o_hbm.at[pl.ds(cb, _SCH)], rows_o)
        pend_o = pltpu.async_copy(rows_o, xo_hbm.at[idx_v.at[c]], sem_o)
    pend_a.wait()
    pend_o.wait()


def _sc_gather_body(h2s_hbm, dest_hbm, aos_hbm, idx_v, rows_v, sem0, sem1):
    wid = lax.axis_index("s") * _NC + lax.axis_index("c")
    base = wid * _PW
    sems = (sem0, sem1)
    pend = []
    for c in range(_PW // _GCH):
        cb = base + c * _GCH
        pltpu.sync_copy(dest_hbm.at[pl.ds(cb, _GCH)], idx_v.at[c])
        pend.append((pltpu.async_copy(h2s_hbm.at[idx_v.at[c]], rows_v.at[c],
                                      sems[c]), c, cb))
    for desc, c, cb in pend:
        desc.wait()
        pltpu.sync_copy(rows_v.at[c], aos_hbm.at[pl.ds(cb, _GCH)])


@functools.lru_cache(maxsize=None)
def _sc_kernels():
    mesh = plsc.VectorSubcoreMesh(core_axis_name="c", subcore_axis_name="s")
    scatter = pl.kernel(
        _sc_scatter_body,
        out_type=(
            jax.ShapeDtypeStruct((P, D), jnp.float32),
            jax.ShapeDtypeStruct((P, D), jnp.float32),
        ),
        mesh=mesh,
        scratch_types=[
            pltpu.VMEM((_SNC, _SCH), jnp.int32),
            pltpu.VMEM((_SCH, D), jnp.float32),
            pltpu.VMEM((_SCH, D), jnp.float32),
            pltpu.SemaphoreType.DMA,
            pltpu.SemaphoreType.DMA,
        ],
    )
    gather = pl.kernel(
        _sc_gather_body,
        out_type=jax.ShapeDtypeStruct((T, H2), jnp.float32),
        mesh=mesh,
        scratch_types=[
            pltpu.VMEM((2, _GCH), jnp.int32),
            pltpu.VMEM((2, _GCH, H2), jnp.float32),
            pltpu.SemaphoreType.DMA,
            pltpu.SemaphoreType.DMA,
        ],
    )
    return scatter, gather


def kernel(u_emb, i_emb, a_emb, o_emb, s,
           mlp_ao_W1, mlp_ao_b1, mlp_ao_W2, mlp_ao_b2,
           mlp_ui_W1, mlp_ui_b1, mlp_ui_W2, mlp_ui_b2):
    bf = jnp.bfloat16
    w1u = mlp_ui_W1[:D].astype(bf)
    w1i = mlp_ui_W1[D:].astype(bf)
    w2ui = mlp_ui_W2.astype(bf)
    b1ui = mlp_ui_b1.reshape(1, H1)
    b2ui = mlp_ui_b2.reshape(1, H2)
    w1a = mlp_ao_W1[:, :D, :].astype(bf)
    w1o = mlp_ao_W1[:, D:, :].astype(bf)
    w2ao = mlp_ao_W2.astype(bf)
    b1ao = mlp_ao_b1
    b2ao = mlp_ao_b2

    s2d = s.reshape(NBLK, BLK)
    a2 = a_emb.reshape(T, D)
    o2 = o_emb.reshape(T, D)

    dest2d, texp = pl.pallas_call(
        _route_body,
        out_shape=(
            jax.ShapeDtypeStruct((NBLK, BLK), jnp.int32),
            jax.ShapeDtypeStruct((1, 128), jnp.int32),
        ),
    )(s2d)

    sc_scatter, sc_gather = _sc_kernels()
    dest = dest2d.reshape(T)
    xa, xo = sc_scatter(a2, o2, dest)

    h2s = pl.pallas_call(
        _group_body,
        grid_spec=pltpu.PrefetchScalarGridSpec(
            num_scalar_prefetch=1,
            grid=(NT,),
            in_specs=[
                pl.BlockSpec((MT, D), lambda t, texp: (t, 0)),
                pl.BlockSpec((MT, D), lambda t, texp: (t, 0)),
                pl.BlockSpec((R, D, H1), lambda t, texp: (0, 0, 0)),
                pl.BlockSpec((R, D, H1), lambda t, texp: (0, 0, 0)),
                pl.BlockSpec((R, H1), lambda t, texp: (0, 0)),
                pl.BlockSpec((R, H1, H2), lambda t, texp: (0, 0, 0)),
                pl.BlockSpec((R, H2), lambda t, texp: (0, 0)),
            ],
            out_specs=pl.BlockSpec((MT, H2), lambda t, texp: (t, 0)),
        ),
        out_shape=jax.ShapeDtypeStruct((P, H2), jnp.float32),
    )(texp[0, :NT], xa, xo, w1a, w1o, b1ao, w2ao, b2ao)

    aos = sc_gather(h2s, dest)

    pred = pl.pallas_call(
        _uidot_body,
        out_shape=jax.ShapeDtypeStruct((B, N), jnp.float32),
    )(u_emb, i_emb, w1u, w1i, b1ui, w2ui, b2ui, aos)
    return pred


# speedup vs baseline: 1.0141x; 1.0141x over previous
import functools

import jax
import jax.numpy as jnp
from jax import lax
from jax.experimental import pallas as pl
from jax.experimental.pallas import tpu as pltpu
from jax.experimental.pallas import tpu_sc as plsc

B, N, D, H1, H2, R = 1024, 8, 512, 512, 256, 8
T = B * N
BLK = 128
NBLK = T // BLK
MT = 512
NT = T // MT + R
P = NT * MT

_NC, _NS = 2, 16
_NW = _NC * _NS
_PW = T // _NW
_SCH = 64
_SNC = _PW // _SCH
_GCH = 128


def _leaky(x):
    return jnp.where(x >= 0, x, 0.01 * x)


def _route_body(s_ref, dest_ref, texp_ref):
    sv = s_ref[...]
    jj = lax.broadcasted_iota(jnp.int32, (BLK, BLK), 0)
    ii = lax.broadcasted_iota(jnp.int32, (BLK, BLK), 1)
    tri = (jj < ii).astype(jnp.float32)
    tj = lax.broadcasted_iota(jnp.int32, (NBLK, NBLK), 1)
    ti = lax.broadcasted_iota(jnp.int32, (NBLK, NBLK), 0)
    tri_blk = (tj < ti).astype(jnp.float32)
    lane = lax.broadcasted_iota(jnp.int32, (1, 128), 1)

    dest_acc = jnp.zeros((NBLK, BLK), jnp.float32)
    texp_acc = jnp.zeros((1, 128), jnp.int32)
    ptile = jnp.float32(0.0)
    for r in range(R):
        oh = (sv == r).astype(jnp.float32)
        excl = jnp.dot(oh, tri, preferred_element_type=jnp.float32)
        counts = jnp.sum(oh, axis=1, keepdims=True)
        base = jnp.dot(tri_blk, counts, preferred_element_type=jnp.float32)
        if r > 0:
            texp_acc = texp_acc + (lane.astype(jnp.float32) >= ptile).astype(jnp.int32)
        dest_acc = dest_acc + oh * (ptile * MT + base + excl)
        total = jnp.sum(counts)
        ptile = ptile + jnp.floor((total + (MT - 1)) * (1.0 / MT))
    dest_ref[...] = dest_acc.astype(jnp.int32)
    texp_ref[...] = texp_acc


def _group_body(texp_ref, xa_ref, xo_ref, w1a_ref, w1o_ref, b1_ref, w2_ref,
                b2_ref, out_ref):
    bf = jnp.bfloat16
    r = texp_ref[pl.program_id(0)]
    h = jnp.dot(xa_ref[...].astype(bf), w1a_ref[r],
                preferred_element_type=jnp.float32)
    h = h + jnp.dot(xo_ref[...].astype(bf), w1o_ref[r],
                    preferred_element_type=jnp.float32)
    h = _leaky(h + b1_ref[r])
    out_ref[...] = _leaky(jnp.dot(h.astype(bf), w2_ref[r],
                                  preferred_element_type=jnp.float32)
                          + b2_ref[r])


def _uidot_body(u_ref, i_ref, w1u_ref, w1i_ref, b1_ref, w2_ref, b2_ref,
                aos_ref, out_ref):
    bf = jnp.bfloat16
    h = jnp.dot(u_ref[...].astype(bf), w1u_ref[...],
                preferred_element_type=jnp.float32)
    h = h + jnp.dot(i_ref[...].astype(bf), w1i_ref[...],
                    preferred_element_type=jnp.float32)
    h = _leaky(h + b1_ref[...])
    ui = _leaky(jnp.dot(h.astype(bf), w2_ref[...],
                        preferred_element_type=jnp.float32) + b2_ref[...])
    aos = aos_ref[...].reshape(B, N, H2)
    out_ref[...] = jnp.sum(aos * ui[:, None, :], axis=-1)


def _sc_scatter_body(a_hbm, o_hbm, dest_hbm, xa_hbm, xo_hbm, idx_v, rows_a,
                     rows_o, sem_a, sem_o):
    wid = lax.axis_index("s") * _NC + lax.axis_index("c")
    base = wid * _PW
    pend_a = pend_o = None
    for c in range(_SNC):
        cb = base + c * _SCH
        pltpu.sync_copy(dest_hbm.at[pl.ds(cb, _SCH)], idx_v.at[c])
        if pend_a is not None:
            pend_a.wait()
        pltpu.sync_copy(a_hbm.at[pl.ds(cb, _SCH)], rows_a)
        pend_a = pltpu.async_copy(rows_a, xa_hbm.at[idx_v.at[c]], sem_a)
        if pend_o is not None:
            pend_o.wait()
        pltpu.sync_copy(o_hbm.at[pl.ds(cb, _SCH)], rows_o)
        pend_o = pltpu.async_copy(rows_o, xo_hbm.at[idx_v.at[c]], sem_o)
    pend_a.wait()
    pend_o.wait()


def _sc_gather_body(h2s_hbm, dest_hbm, aos_hbm, idx_v, rows_v, sem0, sem1):
    wid = lax.axis_index("s") * _NC + lax.axis_index("c")
    base = wid * _PW
    sems = (sem0, sem1)
    pend = []
    for c in range(_PW // _GCH):
        cb = base + c * _GCH
        pltpu.sync_copy(dest_hbm.at[pl.ds(cb, _GCH)], idx_v.at[c])
        pend.append((pltpu.async_copy(h2s_hbm.at[idx_v.at[c]], rows_v.at[c],
                                      sems[c]), c, cb))
    for desc, c, cb in pend:
        desc.wait()
        pltpu.sync_copy(rows_v.at[c], aos_hbm.at[pl.ds(cb, _GCH)])


@functools.lru_cache(maxsize=None)
def _sc_kernels():
    mesh = plsc.VectorSubcoreMesh(core_axis_name="c", subcore_axis_name="s")
    scatter = pl.kernel(
        _sc_scatter_body,
        out_type=(
            jax.ShapeDtypeStruct((P, D), jnp.float32),
            jax.ShapeDtypeStruct((P, D), jnp.float32),
        ),
        mesh=mesh,
        scratch_types=[
            pltpu.VMEM((_SNC, _SCH), jnp.int32),
            pltpu.VMEM((_SCH, D), jnp.float32),
            pltpu.VMEM((_SCH, D), jnp.float32),
            pltpu.SemaphoreType.DMA,
            pltpu.SemaphoreType.DMA,
        ],
    )
    gather = pl.kernel(
        _sc_gather_body,
        out_type=jax.ShapeDtypeStruct((T, H2), jnp.float32),
        mesh=mesh,
        scratch_types=[
            pltpu.VMEM((2, _GCH), jnp.int32),
            pltpu.VMEM((2, _GCH, H2), jnp.float32),
            pltpu.SemaphoreType.DMA,
            pltpu.SemaphoreType.DMA,
        ],
    )
    return scatter, gather


def kernel(u_emb, i_emb, a_emb, o_emb, s,
           mlp_ao_W1, mlp_ao_b1, mlp_ao_W2, mlp_ao_b2,
           mlp_ui_W1, mlp_ui_b1, mlp_ui_W2, mlp_ui_b2):
    bf = jnp.bfloat16
    w1u = mlp_ui_W1[:D].astype(bf)
    w1i = mlp_ui_W1[D:].astype(bf)
    w2ui = mlp_ui_W2.astype(bf)
    b1ui = mlp_ui_b1.reshape(1, H1)
    b2ui = mlp_ui_b2.reshape(1, H2)
    w1a = mlp_ao_W1[:, :D, :].astype(bf)
    w1o = mlp_ao_W1[:, D:, :].astype(bf)
    w2ao = mlp_ao_W2.astype(bf)
    b1ao = mlp_ao_b1
    b2ao = mlp_ao_b2

    s2d = s.reshape(NBLK, BLK)
    a2 = a_emb.reshape(T, D)
    o2 = o_emb.reshape(T, D)

    dest2d, texp = pl.pallas_call(
        _route_body,
        out_shape=(
            jax.ShapeDtypeStruct((NBLK, BLK), jnp.int32),
            jax.ShapeDtypeStruct((1, 128), jnp.int32),
        ),
    )(s2d)

    sc_scatter, sc_gather = _sc_kernels()
    dest = dest2d.reshape(T)
    xa, xo = sc_scatter(a2, o2, dest)

    h2s = pl.pallas_call(
        _group_body,
        grid_spec=pltpu.PrefetchScalarGridSpec(
            num_scalar_prefetch=1,
            grid=(NT,),
            in_specs=[
                pl.BlockSpec((MT, D), lambda t, texp: (t, 0)),
                pl.BlockSpec((MT, D), lambda t, texp: (t, 0)),
                pl.BlockSpec((R, D, H1), lambda t, texp: (0, 0, 0)),
                pl.BlockSpec((R, D, H1), lambda t, texp: (0, 0, 0)),
                pl.BlockSpec((R, H1), lambda t, texp: (0, 0)),
                pl.BlockSpec((R, H1, H2), lambda t, texp: (0, 0, 0)),
                pl.BlockSpec((R, H2), lambda t, texp: (0, 0)),
            ],
            out_specs=pl.BlockSpec((MT, H2), lambda t, texp: (t, 0)),
        ),
        out_shape=jax.ShapeDtypeStruct((P, H2), jnp.float32),
    )(texp[0, :NT], xa, xo, w1a, w1o, b1ao, w2ao, b2ao)

    aos = sc_gather(h2s, dest)

    pred = pl.pallas_call(
        _uidot_body,
        out_shape=jax.ShapeDtypeStruct((B, N), jnp.float32),
    )(u_emb, i_emb, w1u, w1i, b1ui, w2ui, b2ui, aos)
    return pred


# speedup vs baseline: 1.0513x; 1.0367x over previous
import functools

import jax
import jax.numpy as jnp
from jax import lax
from jax.experimental import pallas as pl
from jax.experimental.pallas import tpu as pltpu
from jax.experimental.pallas import tpu_sc as plsc

B, N, D, H1, H2, R = 1024, 8, 512, 512, 256, 8
T = B * N
BLK = 128
NBLK = T // BLK
MT = 512
NT = T // MT + R
P = NT * MT

_NC, _NS = 2, 16
_NW = _NC * _NS
_PW = T // _NW
_SCH = 64
_SNC = _PW // _SCH
_GCH = 128


def _leaky(x):
    return jnp.where(x >= 0, x, 0.01 * x)


def _route_body(s_ref, dest_ref, texp_ref):
    sv = s_ref[...]
    jj = lax.broadcasted_iota(jnp.int32, (BLK, BLK), 0)
    ii = lax.broadcasted_iota(jnp.int32, (BLK, BLK), 1)
    tri = (jj < ii).astype(jnp.float32)
    tj = lax.broadcasted_iota(jnp.int32, (NBLK, NBLK), 1)
    ti = lax.broadcasted_iota(jnp.int32, (NBLK, NBLK), 0)
    tri_blk = (tj < ti).astype(jnp.float32)
    lane = lax.broadcasted_iota(jnp.int32, (1, 128), 1)

    dest_acc = jnp.zeros((NBLK, BLK), jnp.float32)
    texp_acc = jnp.zeros((1, 128), jnp.int32)
    ptile = jnp.float32(0.0)
    for r in range(R):
        oh = (sv == r).astype(jnp.float32)
        excl = jnp.dot(oh, tri, preferred_element_type=jnp.float32)
        counts = jnp.sum(oh, axis=1, keepdims=True)
        base = jnp.dot(tri_blk, counts, preferred_element_type=jnp.float32)
        if r > 0:
            texp_acc = texp_acc + (lane.astype(jnp.float32) >= ptile).astype(jnp.int32)
        dest_acc = dest_acc + oh * (ptile * MT + base + excl)
        total = jnp.sum(counts)
        ptile = ptile + jnp.floor((total + (MT - 1)) * (1.0 / MT))
    dest_ref[...] = dest_acc.astype(jnp.int32)
    texp_ref[...] = texp_acc


def _group_body(texp_ref, xa_ref, xo_ref, w1a_ref, w1o_ref, b1_ref, w2_ref,
                b2_ref, out_ref):
    bf = jnp.bfloat16
    r = texp_ref[pl.program_id(0)]
    h = jnp.dot(xa_ref[...].astype(bf), w1a_ref[r],
                preferred_element_type=jnp.float32)
    h = h + jnp.dot(xo_ref[...].astype(bf), w1o_ref[r],
                    preferred_element_type=jnp.float32)
    h = _leaky(h + b1_ref[r])
    g = _leaky(jnp.dot(h.astype(bf), w2_ref[r],
                       preferred_element_type=jnp.float32) + b2_ref[r])
    gi = jax.lax.bitcast_convert_type(g, jnp.int32)
    rnd = gi + 0x7FFF + ((gi >> 16) & 1)
    lo = (rnd[:, :H2 // 2] >> 16) & 0xFFFF
    hi = rnd[:, H2 // 2:] & jnp.int32(-65536)
    out_ref[...] = hi | lo


def _uidot_body(u_ref, i_ref, w1u_ref, w1i_ref, b1_ref, w2_ref, b2_ref,
                aos_ref, out_ref):
    bf = jnp.bfloat16
    h = jnp.dot(u_ref[...].astype(bf), w1u_ref[...],
                preferred_element_type=jnp.float32)
    h = h + jnp.dot(i_ref[...].astype(bf), w1i_ref[...],
                    preferred_element_type=jnp.float32)
    h = _leaky(h + b1_ref[...])
    ui = _leaky(jnp.dot(h.astype(bf), w2_ref[...],
                        preferred_element_type=jnp.float32) + b2_ref[...])
    packed = aos_ref[...]
    lo = jax.lax.bitcast_convert_type(packed << 16, jnp.float32)
    hi = jax.lax.bitcast_convert_type(packed & jnp.int32(-65536), jnp.float32)
    aos = jnp.concatenate([lo, hi], axis=-1).reshape(B, N, H2)
    out_ref[...] = jnp.sum(aos * ui[:, None, :], axis=-1)


def _sc_scatter_body(a_hbm, o_hbm, dest_hbm, xa_hbm, xo_hbm, idx_v, rows_a,
                     rows_o, sem_a, sem_o):
    wid = lax.axis_index("s") * _NC + lax.axis_index("c")
    base = wid * _PW
    pend_a = pend_o = None
    for c in range(_SNC):
        cb = base + c * _SCH
        pltpu.sync_copy(dest_hbm.at[pl.ds(cb, _SCH)], idx_v.at[c])
        if pend_a is not None:
            pend_a.wait()
        pltpu.sync_copy(a_hbm.at[pl.ds(cb, _SCH)], rows_a)
        pend_a = pltpu.async_copy(rows_a, xa_hbm.at[idx_v.at[c]], sem_a)
        if pend_o is not None:
            pend_o.wait()
        pltpu.sync_copy(o_hbm.at[pl.ds(cb, _SCH)], rows_o)
        pend_o = pltpu.async_copy(rows_o, xo_hbm.at[idx_v.at[c]], sem_o)
    pend_a.wait()
    pend_o.wait()


def _sc_gather_body(h2s_hbm, dest_hbm, aos_hbm, idx_v, rows_v, sem0, sem1):
    wid = lax.axis_index("s") * _NC + lax.axis_index("c")
    base = wid * _PW
    sems = (sem0, sem1)
    pend = []
    for c in range(_PW // _GCH):
        cb = base + c * _GCH
        pltpu.sync_copy(dest_hbm.at[pl.ds(cb, _GCH)], idx_v.at[c])
        pend.append((pltpu.async_copy(h2s_hbm.at[idx_v.at[c]], rows_v.at[c],
                                      sems[c]), c, cb))
    for desc, c, cb in pend:
        desc.wait()
        pltpu.sync_copy(rows_v.at[c], aos_hbm.at[pl.ds(cb, _GCH)])


@functools.lru_cache(maxsize=None)
def _sc_kernels():
    mesh = plsc.VectorSubcoreMesh(core_axis_name="c", subcore_axis_name="s")
    scatter = pl.kernel(
        _sc_scatter_body,
        out_type=(
            jax.ShapeDtypeStruct((P, D), jnp.float32),
            jax.ShapeDtypeStruct((P, D), jnp.float32),
        ),
        mesh=mesh,
        scratch_types=[
            pltpu.VMEM((_SNC, _SCH), jnp.int32),
            pltpu.VMEM((_SCH, D), jnp.float32),
            pltpu.VMEM((_SCH, D), jnp.float32),
            pltpu.SemaphoreType.DMA,
            pltpu.SemaphoreType.DMA,
        ],
    )
    gather = pl.kernel(
        _sc_gather_body,
        out_type=jax.ShapeDtypeStruct((T, H2 // 2), jnp.int32),
        mesh=mesh,
        scratch_types=[
            pltpu.VMEM((2, _GCH), jnp.int32),
            pltpu.VMEM((2, _GCH, H2 // 2), jnp.int32),
            pltpu.SemaphoreType.DMA,
            pltpu.SemaphoreType.DMA,
        ],
    )
    return scatter, gather


def kernel(u_emb, i_emb, a_emb, o_emb, s,
           mlp_ao_W1, mlp_ao_b1, mlp_ao_W2, mlp_ao_b2,
           mlp_ui_W1, mlp_ui_b1, mlp_ui_W2, mlp_ui_b2):
    bf = jnp.bfloat16
    w1u = mlp_ui_W1[:D].astype(bf)
    w1i = mlp_ui_W1[D:].astype(bf)
    w2ui = mlp_ui_W2.astype(bf)
    b1ui = mlp_ui_b1.reshape(1, H1)
    b2ui = mlp_ui_b2.reshape(1, H2)
    w1a = mlp_ao_W1[:, :D, :].astype(bf)
    w1o = mlp_ao_W1[:, D:, :].astype(bf)
    w2ao = mlp_ao_W2.astype(bf)
    b1ao = mlp_ao_b1
    b2ao = mlp_ao_b2

    s2d = s.reshape(NBLK, BLK)
    a2 = a_emb.reshape(T, D)
    o2 = o_emb.reshape(T, D)

    dest2d, texp = pl.pallas_call(
        _route_body,
        out_shape=(
            jax.ShapeDtypeStruct((NBLK, BLK), jnp.int32),
            jax.ShapeDtypeStruct((1, 128), jnp.int32),
        ),
    )(s2d)

    sc_scatter, sc_gather = _sc_kernels()
    dest = dest2d.reshape(T)
    xa, xo = sc_scatter(a2, o2, dest)

    h2s = pl.pallas_call(
        _group_body,
        grid_spec=pltpu.PrefetchScalarGridSpec(
            num_scalar_prefetch=1,
            grid=(NT,),
            in_specs=[
                pl.BlockSpec((MT, D), lambda t, texp: (t, 0)),
                pl.BlockSpec((MT, D), lambda t, texp: (t, 0)),
                pl.BlockSpec((R, D, H1), lambda t, texp: (0, 0, 0)),
                pl.BlockSpec((R, D, H1), lambda t, texp: (0, 0, 0)),
                pl.BlockSpec((R, H1), lambda t, texp: (0, 0)),
                pl.BlockSpec((R, H1, H2), lambda t, texp: (0, 0, 0)),
                pl.BlockSpec((R, H2), lambda t, texp: (0, 0)),
            ],
            out_specs=pl.BlockSpec((MT, H2 // 2), lambda t, texp: (t, 0)),
        ),
        out_shape=jax.ShapeDtypeStruct((P, H2 // 2), jnp.int32),
    )(texp[0, :NT], xa, xo, w1a, w1o, b1ao, w2ao, b2ao)

    aos = sc_gather(h2s, dest)

    pred = pl.pallas_call(
        _uidot_body,
        out_shape=jax.ShapeDtypeStruct((B, N), jnp.float32),
    )(u_emb, i_emb, w1u, w1i, b1ui, w2ui, b2ui, aos)
    return pred
